# baseline (speedup 1.0000x reference)
import jax
import jax.numpy as jnp
from jax import lax
from jax.experimental import pallas as pl
from jax.experimental.pallas import tpu as pltpu
from jax.experimental.pallas import tpu_sc as plsc

VOCAB = 1000000
SEQ = 200
DIM = 64
BATCH = 1024

NC = 2
NS = 16
NW = NC * NS
L = 16

BPW = BATCH // NW
RPW = BPW * SEQ
BPC = 2
CH = BPC * SEQ
NCH = RPW // CH
GSZ = 128
NCOMB = 2 * SEQ
GOFFS = (0, 128, 256, 384)
GLENS = (128, 128, 128, 16)


def _body(word_hbm, type_hbm, tok_hbm, pos_hbm, seg_hbm, out_hbm,
          wordf, typef, idx1d, tok0, tok1, comb_v, seg_v,
          isem, gsem0, gsem1, osem0, osem1):
    wid = lax.axis_index("s") * NC + lax.axis_index("c")
    b0 = wid * BPW
    toks = (tok0, tok1)
    gsems = (gsem0, gsem1)
    osems = (osem0, osem1)
    lanes = lax.broadcasted_iota(jnp.int32, (L,), 0)

    id_descs = []
    for j in range(BPW):
        dst = pl.ds(j * SEQ, SEQ)
        id_descs.append(pltpu.async_copy(word_hbm.at[b0 + j],
                                         wordf.at[dst], isem))
        id_descs.append(pltpu.async_copy(type_hbm.at[b0 + j],
                                         typef.at[dst], isem))

    pltpu.sync_copy(pos_hbm, comb_v.at[pl.ds(0, SEQ)])
    pltpu.sync_copy(pos_hbm, comb_v.at[pl.ds(SEQ, SEQ)])
    pltpu.sync_copy(seg_hbm, seg_v)
    segvec = [[seg_v[t, pl.ds(c * L, L)] for c in range(DIM // L)]
              for t in range(2)]

    @pl.loop(0, SEQ)
    def _comb(s):
        for c in range(DIM // L):
            sl = pl.ds(c * L, L)
            comb_v[s, sl] = comb_v[s, sl] + segvec[0][c]
            comb_v[SEQ + s, sl] = comb_v[SEQ + s, sl] + segvec[1][c]

    for d in id_descs:
        d.wait()

    @pl.loop(0, RPW // L)
    def _cvt(i):
        sl = pl.ds(i * L, L)
        idx1d[sl] = wordf[sl].astype(jnp.int32)

    def fire_gathers(k, buf, sem):
        for off, ln in zip(GOFFS, GLENS):
            pltpu.async_copy(
                tok_hbm.at[idx1d.at[pl.ds(k * CH + off, ln)]],
                buf.at[pl.ds(off, ln)], sem)

    def drain_gathers(buf, sem):
        pltpu.make_async_copy(tok_hbm.at[pl.ds(0, CH)], buf, sem).wait()

    def drain_out(buf, sem):
        for _ in range(BPC):
            pltpu.make_async_copy(buf.at[pl.ds(0, SEQ)], out_hbm.at[0],
                                  sem).wait()

    fire_gathers(0, tok0, gsem0)

    @pl.loop(0, NCH, step=2)
    def _outer(k0):
        for b in range(2):
            k = k0 + b
            buf, sem = toks[b], gsems[b]
            nbuf, nsem = toks[1 - b], gsems[1 - b]

            @pl.when(k + 1 < NCH)
            def _prefetch():
                @pl.when(k >= 1)
                def _drain_prev():
                    drain_out(nbuf, osems[1 - b])

                fire_gathers(k + 1, nbuf, nsem)

            drain_gathers(buf, sem)

            @pl.loop(0, CH // L)
            def _grp(g):
                r0 = g * L
                local = k * CH + r0
                tvec = typef[pl.ds(local, L)].astype(jnp.int32)
                civec = tvec * SEQ + lax.rem(local + lanes, SEQ)
                for l in range(L):
                    ci = civec[l]
                    for c in range(DIM // L):
                        sl = pl.ds(c * L, L)
                        buf[r0 + l, sl] = (buf[r0 + l, sl] * 8.0
                                           + comb_v[ci, sl])

            for j in range(BPC):
                pltpu.async_copy(buf.at[pl.ds(j * SEQ, SEQ)],
                                 out_hbm.at[b0 + k * BPC + j], osems[b])

    drain_out(tok0, osem0)
    drain_out(tok1, osem1)


@jax.jit
def _run(word, typ, token_table, pos, seg):
    mesh = plsc.VectorSubcoreMesh(core_axis_name="c", subcore_axis_name="s")
    return pl.kernel(
        _body,
        out_type=jax.ShapeDtypeStruct((BATCH, SEQ, DIM), jnp.float32),
        mesh=mesh,
        compiler_params=pltpu.CompilerParams(use_tc_tiling_on_sc=False),
        scratch_types=[
            pltpu.VMEM((RPW,), jnp.float32),
            pltpu.VMEM((RPW,), jnp.float32),
            pltpu.VMEM((RPW,), jnp.int32),
            pltpu.VMEM((CH, DIM), jnp.float32),
            pltpu.VMEM((CH, DIM), jnp.float32),
            pltpu.VMEM((NCOMB, DIM), jnp.float32),
            pltpu.VMEM((2, DIM), jnp.float32),
            pltpu.SemaphoreType.DMA,
            pltpu.SemaphoreType.DMA,
            pltpu.SemaphoreType.DMA,
            pltpu.SemaphoreType.DMA,
            pltpu.SemaphoreType.DMA,
        ],
    )(word, typ, token_table, pos, seg)


def kernel(input_word_ids, input_type_ids, token_table, position_table,
           segment_table):
    wf = input_word_ids.astype(jnp.float32)
    tf = input_type_ids.astype(jnp.float32)
    return _run(wf, tf, token_table, position_table, segment_table)


# speedup vs baseline: 1.0034x; 1.0034x over previous
import jax
import jax.numpy as jnp
from jax import lax
from jax.experimental import pallas as pl
from jax.experimental.pallas import tpu as pltpu
from jax.experimental.pallas import tpu_sc as plsc

VOCAB = 1000000
SEQ = 200
DIM = 64
BATCH = 1024

NC = 2
NS = 16
NW = NC * NS
L = 16

BPW = BATCH // NW
RPW = BPW * SEQ
BPC = 2
CH = BPC * SEQ
NCH = RPW // CH
GSZ = 128
NCOMB = 2 * SEQ
GOFFS = (0, 128, 256, 384)
GLENS = (128, 128, 128, 16)


def _body(word_hbm, type_hbm, tok_hbm, pos_hbm, seg_hbm, out_hbm,
          wordf, typef, idx1d, tok0, tok1, comb_v, seg_v,
          isem, gsem0, gsem1, osem0, osem1):
    wid = lax.axis_index("s") * NC + lax.axis_index("c")
    b0 = wid * BPW
    toks = (tok0, tok1)
    gsems = (gsem0, gsem1)
    osems = (osem0, osem1)
    lanes = lax.broadcasted_iota(jnp.int32, (L,), 0)

    id_descs = []
    for j in range(BPW):
        dst = pl.ds(j * SEQ, SEQ)
        id_descs.append(pltpu.async_copy(word_hbm.at[b0 + j, pl.ds(0, SEQ)],
                                         wordf.at[dst], isem))
        id_descs.append(pltpu.async_copy(type_hbm.at[b0 + j, pl.ds(0, SEQ)],
                                         typef.at[dst], isem))

    pltpu.sync_copy(pos_hbm, comb_v.at[pl.ds(0, SEQ)])
    pltpu.sync_copy(pos_hbm, comb_v.at[pl.ds(SEQ, SEQ)])
    pltpu.sync_copy(seg_hbm, seg_v)
    segvec = [[seg_v[t, pl.ds(c * L, L)] for c in range(DIM // L)]
              for t in range(2)]

    @pl.loop(0, SEQ)
    def _comb(s):
        for c in range(DIM // L):
            sl = pl.ds(c * L, L)
            comb_v[s, sl] = comb_v[s, sl] + segvec[0][c]
            comb_v[SEQ + s, sl] = comb_v[SEQ + s, sl] + segvec[1][c]

    for d in id_descs:
        d.wait()

    @pl.loop(0, RPW // L)
    def _cvt(i):
        sl = pl.ds(i * L, L)
        idx1d[sl] = wordf[sl].astype(jnp.int32)

    def fire_gathers(k, buf, sem):
        for off, ln in zip(GOFFS, GLENS):
            pltpu.async_copy(
                tok_hbm.at[idx1d.at[pl.ds(k * CH + off, ln)]],
                buf.at[pl.ds(off, ln)], sem)

    def drain_gathers(buf, sem):
        pltpu.make_async_copy(tok_hbm.at[pl.ds(0, CH)], buf, sem).wait()

    def drain_out(buf, sem):
        for _ in range(BPC):
            pltpu.make_async_copy(buf.at[pl.ds(0, SEQ)], out_hbm.at[0],
                                  sem).wait()

    fire_gathers(0, tok0, gsem0)

    @pl.loop(0, NCH, step=2)
    def _outer(k0):
        for b in range(2):
            k = k0 + b
            buf, sem = toks[b], gsems[b]
            nbuf, nsem = toks[1 - b], gsems[1 - b]

            @pl.when(k + 1 < NCH)
            def _prefetch():
                @pl.when(k >= 1)
                def _drain_prev():
                    drain_out(nbuf, osems[1 - b])

                fire_gathers(k + 1, nbuf, nsem)

            drain_gathers(buf, sem)

            @pl.loop(0, CH // L)
            def _grp(g):
                r0 = g * L
                local = k * CH + r0
                tvec = typef[pl.ds(local, L)].astype(jnp.int32)
                civec = tvec * SEQ + lax.rem(local + lanes, SEQ)
                for l in range(L):
                    ci = civec[l]
                    for c in range(DIM // L):
                        sl = pl.ds(c * L, L)
                        buf[r0 + l, sl] = (buf[r0 + l, sl] * 8.0
                                           + comb_v[ci, sl])

            for j in range(BPC):
                pltpu.async_copy(buf.at[pl.ds(j * SEQ, SEQ)],
                                 out_hbm.at[b0 + k * BPC + j], osems[b])

    drain_out(tok0, osem0)
    drain_out(tok1, osem1)


@jax.jit
def _run(word, typ, token_table, pos, seg):
    mesh = plsc.VectorSubcoreMesh(core_axis_name="c", subcore_axis_name="s")
    return pl.kernel(
        _body,
        out_type=jax.ShapeDtypeStruct((BATCH, SEQ, DIM), jnp.float32),
        mesh=mesh,
        compiler_params=pltpu.CompilerParams(use_tc_tiling_on_sc=False),
        scratch_types=[
            pltpu.VMEM((RPW,), jnp.float32),
            pltpu.VMEM((RPW,), jnp.float32),
            pltpu.VMEM((RPW,), jnp.int32),
            pltpu.VMEM((CH, DIM), jnp.float32),
            pltpu.VMEM((CH, DIM), jnp.float32),
            pltpu.VMEM((NCOMB, DIM), jnp.float32),
            pltpu.VMEM((2, DIM), jnp.float32),
            pltpu.SemaphoreType.DMA,
            pltpu.SemaphoreType.DMA,
            pltpu.SemaphoreType.DMA,
            pltpu.SemaphoreType.DMA,
            pltpu.SemaphoreType.DMA,
        ],
    )(word, typ, token_table, pos, seg)


def kernel(input_word_ids, input_type_ids, token_table, position_table,
           segment_table):
    wf = jnp.pad(input_word_ids.astype(jnp.float32), ((0, 0), (0, 56)))
    tf = jnp.pad(input_type_ids.astype(jnp.float32), ((0, 0), (0, 56)))
    return _run(wf, tf, token_table, position_table, segment_table)


# speedup vs baseline: 1.0179x; 1.0145x over previous
import jax
import jax.numpy as jnp
from jax import lax
from jax.experimental import pallas as pl
from jax.experimental.pallas import tpu as pltpu
from jax.experimental.pallas import tpu_sc as plsc

VOCAB = 1000000
SEQ = 200
DIM = 64
BATCH = 1024

NC = 2
NS = 16
NW = NC * NS
L = 16

BPW = BATCH // NW
RPW = BPW * SEQ
BPC = 2
CH = BPC * SEQ
NCH = RPW // CH
NCOMB = 2 * SEQ
GOFFS = (0, 128, 256, 384)
GLENS = (128, 128, 128, 16)


def _body(word_hbm, ci_hbm, tok_hbm, pos_hbm, seg_hbm, out_hbm,
          idx1d, civ, tok0, tok1, comb_v, seg_v,
          isem, gsem0, gsem1, osem0, osem1):
    wid = lax.axis_index("s") * NC + lax.axis_index("c")
    b0 = wid * BPW
    toks = (tok0, tok1)
    gsems = (gsem0, gsem1)
    osems = (osem0, osem1)

    id_descs = []
    for j in range(BPW):
        dst = pl.ds(j * SEQ, SEQ)
        id_descs.append(pltpu.async_copy(word_hbm.at[b0 + j],
                                         idx1d.at[dst], isem))
        id_descs.append(pltpu.async_copy(ci_hbm.at[b0 + j],
                                         civ.at[dst], isem))

    pltpu.sync_copy(pos_hbm, comb_v.at[pl.ds(0, SEQ)])
    pltpu.sync_copy(pos_hbm, comb_v.at[pl.ds(SEQ, SEQ)])
    pltpu.sync_copy(seg_hbm, seg_v)
    segvec = [[seg_v[t, pl.ds(c * L, L)] for c in range(DIM // L)]
              for t in range(2)]

    @pl.loop(0, SEQ)
    def _comb(s):
        for c in range(DIM // L):
            sl = pl.ds(c * L, L)
            comb_v[s, sl] = comb_v[s, sl] + segvec[0][c]
            comb_v[SEQ + s, sl] = comb_v[SEQ + s, sl] + segvec[1][c]

    for d in id_descs:
        d.wait()

    def fire_gathers(k, buf, sem):
        for off, ln in zip(GOFFS, GLENS):
            pltpu.async_copy(
                tok_hbm.at[idx1d.at[pl.ds(k * CH + off, ln)]],
                buf.at[pl.ds(off, ln)], sem)

    def drain_gathers(buf, sem):
        pltpu.make_async_copy(tok_hbm.at[pl.ds(0, CH)], buf, sem).wait()

    def drain_out(buf, sem):
        for _ in range(BPC):
            pltpu.make_async_copy(buf.at[pl.ds(0, SEQ)], out_hbm.at[0],
                                  sem).wait()

    fire_gathers(0, tok0, gsem0)

    @pl.loop(0, NCH, step=2)
    def _outer(k0):
        for b in range(2):
            k = k0 + b
            buf, sem = toks[b], gsems[b]
            nbuf, nsem = toks[1 - b], gsems[1 - b]

            @pl.when(k + 1 < NCH)
            def _prefetch():
                @pl.when(k >= 1)
                def _drain_prev():
                    drain_out(nbuf, osems[1 - b])

                fire_gathers(k + 1, nbuf, nsem)

            drain_gathers(buf, sem)

            @pl.loop(0, CH // L, unroll=2)
            def _grp(g):
                r0 = g * L
                civec = civ[pl.ds(k * CH + r0, L)]
                for l in range(L):
                    ci = civec[l]
                    for c in range(DIM // L):
                        sl = pl.ds(c * L, L)
                        buf[r0 + l, sl] = (buf[r0 + l, sl] * 8.0
                                           + comb_v[ci, sl])

            for j in range(BPC):
                pltpu.async_copy(buf.at[pl.ds(j * SEQ, SEQ)],
                                 out_hbm.at[b0 + k * BPC + j], osems[b])

    drain_out(tok0, osem0)
    drain_out(tok1, osem1)


@jax.jit
def _run(word, ci, token_table, pos, seg):
    mesh = plsc.VectorSubcoreMesh(core_axis_name="c", subcore_axis_name="s")
    return pl.kernel(
        _body,
        out_type=jax.ShapeDtypeStruct((BATCH, SEQ, DIM), jnp.float32),
        mesh=mesh,
        compiler_params=pltpu.CompilerParams(use_tc_tiling_on_sc=False),
        scratch_types=[
            pltpu.VMEM((RPW,), jnp.int32),
            pltpu.VMEM((RPW,), jnp.int32),
            pltpu.VMEM((CH, DIM), jnp.float32),
            pltpu.VMEM((CH, DIM), jnp.float32),
            pltpu.VMEM((NCOMB, DIM), jnp.float32),
            pltpu.VMEM((2, DIM), jnp.float32),
            pltpu.SemaphoreType.DMA,
            pltpu.SemaphoreType.DMA,
            pltpu.SemaphoreType.DMA,
            pltpu.SemaphoreType.DMA,
            pltpu.SemaphoreType.DMA,
        ],
    )(word, ci, token_table, pos, seg)


def kernel(input_word_ids, input_type_ids, token_table, position_table,
           segment_table):
    ci = input_type_ids * SEQ + jnp.arange(SEQ, dtype=jnp.int32)[None, :]
    return _run(input_word_ids, ci, token_table, position_table,
                segment_table)
